# Initial kernel scaffold; baseline (speedup 1.0000x reference)
import jax
import jax.numpy as jnp
from jax.experimental import pallas as pl


def kernel(x, edge_index1, edge_index2, edge_index3, weight_b, W1, b1, W2, b2):
    raise NotImplementedError("write your pallas kernel here")


# speedup vs baseline: 8.1150x; 8.1150x over previous
import functools

import jax
import jax.numpy as jnp
from jax import lax
from jax.experimental import pallas as pl
from jax.experimental.pallas import tpu as pltpu
from jax.experimental.pallas import tpu_sc as plsc

NC = 2
NS = 16
L = 16
CH = 128


def _spmm_body(npad, e, rounds, nchunk, s1, s2, s3, e1, e2, e3, part,
               acc, idx2, rows, sem):
    c = lax.axis_index("c")
    s = lax.axis_index("s")
    w = c * NS + s
    rpt = npad // NS

    @pl.loop(0, CH)
    def _zero_rows(i):
        for j in range(rows.shape[1] // L):
            rows[i, j * L:(j + 1) * L] = jnp.zeros((L,), jnp.float32)

    for k in range(rpt // CH):
        pltpu.sync_copy(rows, acc.at[pl.ds(s * rpt + k * CH, CH)])
    plsc.subcore_barrier()

    for r, e_ref, s_ref in ((0, e1, s1), (1, e2, s2), (2, e3, s3)):
        for d in (0, 1):
            @pl.loop(0, rounds)
            def _chunk(i, _r=r, _d=d, _e=e_ref, _s=s_ref):
                chunk = w + (NC * NS) * i

                @pl.when(chunk < nchunk)
                def _():
                    off = chunk * CH
                    pltpu.sync_copy(_e.at[:, pl.ds(off, CH)], idx2)
                    pltpu.async_copy(_s.at[idx2.at[_d]], rows, sem).wait()
                    pltpu.sync_copy(rows, acc.at[idx2.at[1 - _d]], add=True)

    plsc.subcore_barrier()
    pltpu.sync_copy(acc.at[pl.ds(s * rpt, rpt)],
                    part.at[c, pl.ds(s * rpt, rpt)])


def _sc_spmm(s1, s2, s3, e1, e2, e3):
    n, d_model = s1.shape
    e = e1.shape[1]
    nchunk = e // CH
    rounds = pl.cdiv(nchunk, NC * NS)
    npad = ((n + NS * CH - 1) // (NS * CH)) * (NS * CH)
    mesh = plsc.VectorSubcoreMesh(core_axis_name="c", subcore_axis_name="s")
    body = functools.partial(_spmm_body, npad, e, rounds, nchunk)
    return pl.kernel(
        body,
        out_type=jax.ShapeDtypeStruct((NC, npad, d_model), jnp.float32),
        mesh=mesh,
        scratch_types=[
            pltpu.VMEM_SHARED((npad, d_model), jnp.float32),
            pltpu.VMEM((2, CH), jnp.int32),
            pltpu.VMEM((CH, d_model), jnp.float32),
            pltpu.SemaphoreType.DMA,
        ],
    )(s1, s2, s3, e1, e2, e3)


def _tc_scaled_support_body(x_ref, w_ref, wb_ref, s1_ref, s2_ref, s3_ref):
    sup = jnp.dot(x_ref[...], w_ref[...], preferred_element_type=jnp.float32)
    s1_ref[...] = wb_ref[0, 0] * sup
    s2_ref[...] = wb_ref[1, 0] * sup
    s3_ref[...] = wb_ref[2, 0] * sup


def _tc_scaled_support(x, w, wb, bm):
    n, d_model = x.shape
    grid = (n // bm,)
    blk = pl.BlockSpec((bm, d_model), lambda i: (i, 0))
    return pl.pallas_call(
        _tc_scaled_support_body,
        grid=grid,
        in_specs=[blk,
                  pl.BlockSpec((d_model, d_model), lambda i: (0, 0)),
                  pl.BlockSpec(memory_space=pltpu.SMEM)],
        out_specs=[blk, blk, blk],
        out_shape=[jax.ShapeDtypeStruct((n, d_model), jnp.float32)] * 3,
    )(x, w, wb)


def _tc_merge_support_body(p_ref, b_ref, w_ref, wb_ref,
                           u_ref, s1_ref, s2_ref, s3_ref):
    u = p_ref[0] + p_ref[1] + b_ref[...]
    u_ref[...] = u
    sup = jnp.dot(u, w_ref[...], preferred_element_type=jnp.float32)
    s1_ref[...] = wb_ref[0, 0] * sup
    s2_ref[...] = wb_ref[1, 0] * sup
    s3_ref[...] = wb_ref[2, 0] * sup


def _tc_merge_support(p, b, w, wb, bm, n):
    _, _, d_model = p.shape
    grid = (n // bm,)
    blk = pl.BlockSpec((bm, d_model), lambda i: (i, 0))
    return pl.pallas_call(
        _tc_merge_support_body,
        grid=grid,
        in_specs=[pl.BlockSpec((2, bm, d_model), lambda i: (0, i, 0)),
                  pl.BlockSpec((1, d_model), lambda i: (0, 0)),
                  pl.BlockSpec((d_model, d_model), lambda i: (0, 0)),
                  pl.BlockSpec(memory_space=pltpu.SMEM)],
        out_specs=[blk, blk, blk, blk],
        out_shape=[jax.ShapeDtypeStruct((n, d_model), jnp.float32)] * 4,
    )(p, b.reshape(1, d_model), w, wb)


def _tc_final_body(u1_ref, q_ref, b_ref, out_ref):
    out_ref[...] = 0.5 * (u1_ref[...] + q_ref[0] + q_ref[1] + b_ref[...])


def _tc_final(u1, q, b, bm):
    n, d_model = u1.shape
    grid = (n // bm,)
    blk = pl.BlockSpec((bm, d_model), lambda i: (i, 0))
    return pl.pallas_call(
        _tc_final_body,
        grid=grid,
        in_specs=[blk,
                  pl.BlockSpec((2, bm, d_model), lambda i: (0, i, 0)),
                  pl.BlockSpec((1, d_model), lambda i: (0, 0))],
        out_specs=blk,
        out_shape=jax.ShapeDtypeStruct((n, d_model), jnp.float32),
    )(u1, q, b.reshape(1, d_model))


def kernel(x, edge_index1, edge_index2, edge_index3, weight_b, W1, b1, W2, b2):
    n, d_model = x.shape
    bm = 1000
    e1 = edge_index1.astype(jnp.int32)
    e2 = edge_index2.astype(jnp.int32)
    e3 = edge_index3.astype(jnp.int32)

    s1, s2, s3 = _tc_scaled_support(x, W1, weight_b, bm)
    p = _sc_spmm(s1, s2, s3, e1, e2, e3)
    u1, t1, t2, t3 = _tc_merge_support(p, b1, W2, weight_b, bm, n)
    q = _sc_spmm(t1, t2, t3, e1, e2, e3)
    return _tc_final(u1, q, b2, bm)
